# Initial kernel scaffold; baseline (speedup 1.0000x reference)
import jax
import jax.numpy as jnp
from jax import lax
from jax.experimental import pallas as pl
from jax.experimental.pallas import tpu as pltpu
from jax.experimental.pallas import tpu_sc as plsc

VOCAB = 49408
HIDDEN = 768
MAX_POS = 77
BATCH = 1024
SEQ = 77

NC = 2
NS = 16
NW = NC * NS
ROWS = BATCH * SEQ
RPW = ROWS // NW
CH = 11
CPW = RPW // CH
CHPS = SEQ // CH
NBUF = 2
VPR = HIDDEN // 16


def _body(idx_hbm, token_hbm, pos_hbm, out_hbm,
          idx_v, pos_v, inbuf, outbuf,
          gsem0, gsem1, wsem0, wsem1):
  gsems = [gsem0, gsem1]
  wsems = [wsem0, wsem1]
  wid = lax.axis_index("s") * NC + lax.axis_index("c")
  cbase = wid * CPW

  pltpu.sync_copy(pos_hbm, pos_v)
  pltpu.sync_copy(idx_hbm.at[pl.ds(cbase, CPW)], idx_v)

  def start_gather(c, b):
    return pltpu.async_copy(token_hbm.at[idx_v.at[c]], inbuf.at[b], gsems[b])

  for b in range(NBUF):
    start_gather(b, b)

  @pl.loop(0, CPW, step=NBUF)
  def _chunks(c0):
    for b in range(NBUF):
      c = c0 + b
      pltpu.make_async_copy(token_hbm.at[idx_v.at[c]], inbuf.at[b],
                            gsems[b]).wait()

      @pl.when(c0 >= NBUF)
      def _():
        row_prev = (cbase + c - NBUF) * CH
        pltpu.make_async_copy(outbuf.at[b],
                              out_hbm.at[pl.ds(row_prev, CH)],
                              wsems[b]).wait()

      prow0 = lax.rem(c, CHPS) * CH

      @pl.loop(0, CH)
      def _rows(r):
        for v in range(VPR):
          sl = pl.ds(v * 16, 16)
          outbuf[b, r, sl] = inbuf[b, r, sl] + pos_v[prow0 + r, sl]

      row0 = (cbase + c) * CH
      pltpu.async_copy(outbuf.at[b], out_hbm.at[pl.ds(row0, CH)], wsems[b])

      @pl.when(c0 + NBUF < CPW)
      def _():
        start_gather(c + NBUF, b)

  for b in range(NBUF):
    row0 = (cbase + CPW - NBUF + b) * CH
    pltpu.make_async_copy(outbuf.at[b], out_hbm.at[pl.ds(row0, CH)],
                          wsems[b]).wait()


@jax.jit
def _run(idx2d, token_table, pos_flat):
  mesh = plsc.VectorSubcoreMesh(core_axis_name="c", subcore_axis_name="s",
                                num_cores=NC, num_subcores=NS)
  f = pl.kernel(
      _body,
      out_type=jax.ShapeDtypeStruct((ROWS, HIDDEN), jnp.float32),
      mesh=mesh,
      scratch_types=[
          pltpu.VMEM((CPW, CH), jnp.int32),
          pltpu.VMEM((MAX_POS, HIDDEN), jnp.float32),
          pltpu.VMEM((NBUF, CH, HIDDEN), jnp.float32),
          pltpu.VMEM((NBUF, CH, HIDDEN), jnp.float32),
          pltpu.SemaphoreType.DMA,
          pltpu.SemaphoreType.DMA,
          pltpu.SemaphoreType.DMA,
          pltpu.SemaphoreType.DMA,
      ],
  )
  return f(idx2d, token_table, pos_flat)


def kernel(input_ids, position_ids, token_table, pos_table):
  del position_ids
  idx2d = input_ids.astype(jnp.int32).reshape(ROWS // CH, CH)
  out = _run(idx2d, token_table, pos_table.reshape(MAX_POS * HIDDEN))
  return out.reshape(BATCH, SEQ, HIDDEN)


# speedup vs baseline: 1.0933x; 1.0933x over previous
import jax
import jax.numpy as jnp
from jax import lax
from jax.experimental import pallas as pl
from jax.experimental.pallas import tpu as pltpu
from jax.experimental.pallas import tpu_sc as plsc

VOCAB = 49408
HIDDEN = 768
MAX_POS = 77
BATCH = 1024
SEQ = 77

NC = 2
NS = 16
NW = NC * NS
ROWS = BATCH * SEQ
RPW = ROWS // NW
CH = 16
CPW = RPW // CH
NBUF = 2
VPR = HIDDEN // 16
CHW = CH * HIDDEN


def _body(idx_hbm, token_hbm, pos_hbm, out_hbm,
          idx_v, pos_v, inbuf, outbuf,
          gsem0, gsem1, wsem0, wsem1):
  gsems = [gsem0, gsem1]
  wsems = [wsem0, wsem1]
  wid = lax.axis_index("s") * NC + lax.axis_index("c")
  rbase = wid * RPW

  pltpu.sync_copy(pos_hbm, pos_v)
  pltpu.sync_copy(idx_hbm.at[pl.ds(rbase, RPW)], idx_v)

  def gather_copy(c, b):
    idxvec = idx_v[pl.ds(c * CH, CH)]
    return pltpu.make_async_copy(token_hbm.at[idxvec], inbuf.at[b], gsems[b])

  for b in range(NBUF):
    gather_copy(b, b).start()

  @pl.loop(0, CPW, step=NBUF)
  def _chunks(c0):
    for b in range(NBUF):
      c = c0 + b
      gather_copy(c, b).wait()

      @pl.when(c0 >= NBUF)
      def _():
        eprev = (rbase + (c - NBUF) * CH) * HIDDEN
        pltpu.make_async_copy(outbuf.at[b],
                              out_hbm.at[pl.ds(eprev, CHW)],
                              wsems[b]).wait()

      p0 = lax.rem(c * CH, SEQ)

      @pl.loop(0, CH)
      def _rows(r):
        prow = lax.rem(p0 + r, SEQ)
        ob = r * HIDDEN
        for v in range(VPR):
          sl = pl.ds(v * 16, 16)
          outbuf[b, pl.ds(ob + v * 16, 16)] = (
              inbuf[b, r, sl] + pos_v[prow, sl])

      e0 = (rbase + c * CH) * HIDDEN
      pltpu.async_copy(outbuf.at[b], out_hbm.at[pl.ds(e0, CHW)], wsems[b])

      @pl.when(c0 + NBUF < CPW)
      def _():
        gather_copy(c + NBUF, b).start()

  for b in range(NBUF):
    e0 = (rbase + (CPW - NBUF + b) * CH) * HIDDEN
    pltpu.make_async_copy(outbuf.at[b], out_hbm.at[pl.ds(e0, CHW)],
                          wsems[b]).wait()


@jax.jit
def _run(ids_flat, token_table, pos_table):
  mesh = plsc.VectorSubcoreMesh(core_axis_name="c", subcore_axis_name="s",
                                num_cores=NC, num_subcores=NS)
  f = pl.kernel(
      _body,
      out_type=jax.ShapeDtypeStruct((ROWS * HIDDEN,), jnp.float32),
      mesh=mesh,
      scratch_types=[
          pltpu.VMEM((RPW,), jnp.int32),
          pltpu.VMEM((MAX_POS, HIDDEN), jnp.float32),
          pltpu.VMEM((NBUF, CH, HIDDEN), jnp.float32),
          pltpu.VMEM((NBUF, CHW), jnp.float32),
          pltpu.SemaphoreType.DMA,
          pltpu.SemaphoreType.DMA,
          pltpu.SemaphoreType.DMA,
          pltpu.SemaphoreType.DMA,
      ],
  )
  return f(ids_flat, token_table, pos_table)


def kernel(input_ids, position_ids, token_table, pos_table):
  del position_ids
  ids_flat = input_ids.astype(jnp.int32).reshape(ROWS)
  out = _run(ids_flat, token_table, pos_table)
  return out.reshape(BATCH, SEQ, HIDDEN)


# speedup vs baseline: 2.7706x; 2.5342x over previous
import jax
import jax.numpy as jnp
from jax import lax
from jax.experimental import pallas as pl
from jax.experimental.pallas import tpu as pltpu
from jax.experimental.pallas import tpu_sc as plsc

VOCAB = 49408
HIDDEN = 768
MAX_POS = 77
BATCH = 1024
SEQ = 77
SEQP = 80

NC = 2
NS = 16
NW = NC * NS
SPW = BATCH // NW
CH = 16
NRING = 4
TS0 = 64
TROWS = 13
VPR = HIDDEN // 16


def _body(idx_hbm, token_hbm, pos_hbm, out_hbm,
          idx_v, pos_v, buf, tailbuf,
          gsem0, gsem1, gsem2, gsem3, wsem0, wsem1, wsem2, wsem3,
          tgsem, twsem):
  gsems = [gsem0, gsem1, gsem2, gsem3]
  wsems = [wsem0, wsem1, wsem2, wsem3]
  wid = lax.axis_index("s") * NC + lax.axis_index("c")
  sbase = wid * SPW

  pltpu.sync_copy(pos_hbm, pos_v)
  pltpu.sync_copy(idx_hbm.at[pl.ds(sbase, SPW)], idx_v)

  def gather(b_local, j):
    idxvec = idx_v[b_local, pl.ds(j * CH, CH)]
    return pltpu.make_async_copy(token_hbm.at[idxvec], buf.at[j], gsems[j])

  def tail_gather(b_local):
    idxvec = idx_v[b_local, pl.ds(TS0, CH)]
    return pltpu.make_async_copy(token_hbm.at[idxvec], buf.at[0], tgsem)

  def wb(b_local, j):
    return pltpu.make_async_copy(
        buf.at[j], out_hbm.at[sbase + b_local, pl.ds(j * CH, CH)], wsems[j])

  def tail_wb(b_local):
    return pltpu.make_async_copy(
        tailbuf, out_hbm.at[sbase + b_local, pl.ds(TS0, TROWS)], twsem)

  def add_pos(j):
    @pl.loop(0, CH)
    def _rows(r):
      for v in range(VPR):
        sl = pl.ds(v * 16, 16)
        plsc.addupdate(buf.at[j, r, sl], pos_v[j * CH + r, sl])

  gather(0, 0).start()
  gather(0, 1).start()

  @pl.loop(0, SPW)
  def _seqs(b_local):
    gather(b_local, 0).wait()
    add_pos(0)
    wb(b_local, 0).start()

    @pl.when(b_local >= 1)
    def _():
      wb(b_local - 1, 2).wait()
      gather(b_local, 2).start()

    @pl.when(b_local == 0)
    def _():
      gather(0, 2).start()

    gather(b_local, 1).wait()
    add_pos(1)
    wb(b_local, 1).start()

    @pl.when(b_local >= 1)
    def _():
      wb(b_local - 1, 3).wait()
      gather(b_local, 3).start()

    @pl.when(b_local == 0)
    def _():
      gather(0, 3).start()

    gather(b_local, 2).wait()
    add_pos(2)
    wb(b_local, 2).start()
    wb(b_local, 0).wait()
    tail_gather(b_local).start()

    gather(b_local, 3).wait()
    add_pos(3)
    wb(b_local, 3).start()

    @pl.when(b_local <= SPW - 2)
    def _():
      wb(b_local, 1).wait()
      gather(b_local + 1, 1).start()

    tail_gather(b_local).wait()

    @pl.when(b_local >= 1)
    def _():
      tail_wb(b_local - 1).wait()

    @pl.loop(0, TROWS)
    def _trows(r):
      for v in range(VPR):
        sl = pl.ds(v * 16, 16)
        tailbuf[r, sl] = buf[0, r, sl] + pos_v[TS0 + r, sl]

    tail_wb(b_local).start()

    @pl.when(b_local <= SPW - 2)
    def _():
      gather(b_local + 1, 0).start()

  wb(SPW - 1, 1).wait()
  wb(SPW - 1, 2).wait()
  wb(SPW - 1, 3).wait()
  tail_wb(SPW - 1).wait()


@jax.jit
def _run(ids_padded, token_table, pos_table):
  mesh = plsc.VectorSubcoreMesh(core_axis_name="c", subcore_axis_name="s",
                                num_cores=NC, num_subcores=NS)
  f = pl.kernel(
      _body,
      out_type=jax.ShapeDtypeStruct((BATCH, SEQ, HIDDEN), jnp.float32),
      mesh=mesh,
      scratch_types=[
          pltpu.VMEM((SPW, SEQP), jnp.int32),
          pltpu.VMEM((MAX_POS, HIDDEN), jnp.float32),
          pltpu.VMEM((NRING, CH, HIDDEN), jnp.float32),
          pltpu.VMEM((TROWS, HIDDEN), jnp.float32),
          pltpu.SemaphoreType.DMA,
          pltpu.SemaphoreType.DMA,
          pltpu.SemaphoreType.DMA,
          pltpu.SemaphoreType.DMA,
          pltpu.SemaphoreType.DMA,
          pltpu.SemaphoreType.DMA,
          pltpu.SemaphoreType.DMA,
          pltpu.SemaphoreType.DMA,
          pltpu.SemaphoreType.DMA,
          pltpu.SemaphoreType.DMA,
      ],
  )
  return f(ids_padded, token_table, pos_table)


def kernel(input_ids, position_ids, token_table, pos_table):
  del position_ids
  ids = input_ids.astype(jnp.int32)
  ids_padded = jnp.concatenate([ids, ids[:, -3:]], axis=1)
  return _run(ids_padded, token_table, pos_table)


# speedup vs baseline: 6.2676x; 2.2622x over previous
import jax
import jax.numpy as jnp
from jax import lax
from jax.experimental import pallas as pl
from jax.experimental.pallas import tpu as pltpu
from jax.experimental.pallas import tpu_sc as plsc

VOCAB = 49408
HIDDEN = 768
MAX_POS = 77
BATCH = 1024
SEQ = 77

NC = 2
NS = 16
NW = NC * NS
ROWS = BATCH * SEQ
RPW = ROWS // NW
CH = 16
CPW = RPW // CH
NBUF = 7
LEAD = 3
VPR = HIDDEN // 16
NPOS = 4


def _body(idx_hbm, token_hbm, pos_hbm, out_hbm,
          idx_v, pos_v, buf,
          gsem0, gsem1, gsem2, gsem3, gsem4, gsem5, gsem6,
          wsem0, wsem1, wsem2, wsem3, wsem4, wsem5, wsem6):
  gsems = [gsem0, gsem1, gsem2, gsem3, gsem4, gsem5, gsem6]
  wsems = [wsem0, wsem1, wsem2, wsem3, wsem4, wsem5, wsem6]
  wid = lax.axis_index("s") * NC + lax.axis_index("c")
  rbase = wid * RPW
  s_lo = rbase // BATCH

  pltpu.sync_copy(pos_hbm.at[pl.ds(s_lo * HIDDEN, NPOS * HIDDEN)], pos_v)
  pltpu.sync_copy(idx_hbm.at[pl.ds(rbase, RPW)], idx_v)

  def gather(c, slot):
    idxvec = idx_v[pl.ds(c * CH, CH)]
    return pltpu.make_async_copy(token_hbm.at[idxvec], buf.at[slot],
                                 gsems[slot])

  def wb(c, slot):
    return pltpu.make_async_copy(
        buf.at[slot], out_hbm.at[pl.ds(rbase + c * CH, CH)], wsems[slot])

  for c in range(LEAD):
    gather(c, c).start()

  @pl.loop(0, CPW, step=NBUF)
  def _chunks(c0):
    for b in range(NBUF):
      c = c0 + b
      gather(c, b).wait()

      srel = (rbase + c * CH) // BATCH - s_lo

      @pl.loop(0, VPR)
      def _cols(v):
        pv = pos_v[pl.ds(srel * HIDDEN + v * 16, 16)]
        for r in range(CH):
          plsc.addupdate(buf.at[b, r, pl.ds(v * 16, 16)], pv)

      wb(c, b).start()

      slot_n = (b + LEAD) % NBUF

      @pl.when(c + LEAD < CPW)
      def _():
        @pl.when(c + LEAD - NBUF >= 0)
        def _():
          wb(c + LEAD - NBUF, slot_n).wait()

        gather(c + LEAD, slot_n).start()

  for c in range(CPW - NBUF, CPW):
    wb(c, c % NBUF).wait()


@jax.jit
def _run(ids_t_flat, token_table, pos_flat):
  mesh = plsc.VectorSubcoreMesh(core_axis_name="c", subcore_axis_name="s",
                                num_cores=NC, num_subcores=NS)
  f = pl.kernel(
      _body,
      out_type=jax.ShapeDtypeStruct((ROWS, HIDDEN), jnp.float32),
      mesh=mesh,
      scratch_types=[
          pltpu.VMEM((RPW,), jnp.int32),
          pltpu.VMEM((NPOS * HIDDEN,), jnp.float32),
          pltpu.VMEM((NBUF, CH, HIDDEN), jnp.float32),
          pltpu.SemaphoreType.DMA,
          pltpu.SemaphoreType.DMA,
          pltpu.SemaphoreType.DMA,
          pltpu.SemaphoreType.DMA,
          pltpu.SemaphoreType.DMA,
          pltpu.SemaphoreType.DMA,
          pltpu.SemaphoreType.DMA,
          pltpu.SemaphoreType.DMA,
          pltpu.SemaphoreType.DMA,
          pltpu.SemaphoreType.DMA,
          pltpu.SemaphoreType.DMA,
          pltpu.SemaphoreType.DMA,
          pltpu.SemaphoreType.DMA,
          pltpu.SemaphoreType.DMA,
      ],
  )
  return f(ids_t_flat, token_table, pos_flat)


def kernel(input_ids, position_ids, token_table, pos_table):
  del position_ids
  ids_t_flat = input_ids.astype(jnp.int32).T.reshape(ROWS)
  pos_flat = jnp.concatenate(
      [pos_table.reshape(MAX_POS * HIDDEN),
       jnp.zeros((NPOS * HIDDEN,), jnp.float32)])
  out = _run(ids_t_flat, token_table, pos_flat)
  return out.reshape(SEQ, BATCH, HIDDEN).transpose(1, 0, 2)
